# baseline (speedup 1.0000x reference)
import functools

import jax
import jax.numpy as jnp
from jax import lax
from jax.experimental import pallas as pl
from jax.experimental.pallas import tpu as pltpu
from jax.experimental.pallas import tpu_sc as plsc

DIM = 128
NCOLS = 2 * DIM
NC = 2
NS = 16
NW = NC * NS
CHUNK = 64


def _vsqrt(x):
    i = lax.bitcast_convert_type(x, jnp.int32)
    y = lax.bitcast_convert_type(0x5F3759DF - (i >> 1), jnp.float32)
    for _ in range(3):
        y = y * (1.5 - 0.5 * x * y * y)
    return x * y


def _relu(v):
    return jnp.maximum(v, 0.0)


def _sc_body(cls_hbm, rel_hbm,
             a10, a11,
             a20, a21, a22,
             a30, a31, a32,
             a40, a41, a42,
             a50, a51,
             a60, a61, a62,
             out_hbm,
             ix1, ix2, ix3, ix4, ix5, ix6,
             ca, cb, ce, rb, outv, sem0, sem1):
    wid = lax.axis_index("s") * NC + lax.axis_index("c")
    n_items = a10.shape[0]
    per_w = n_items // NW
    per_w_neg = a60.shape[0] // NW
    nch = per_w // CHUNK
    nch_neg = per_w_neg // CHUNK
    f0 = jnp.zeros((16,), jnp.float32)
    lane = lax.iota(jnp.int32, 16)
    sems = (sem0, sem1)

    b0 = wid * per_w
    bn = wid * per_w_neg
    for ref, cols_ in ((ix1, (a10, a11)), (ix2, (a20, a21, a22)),
                       (ix3, (a30, a31, a32)), (ix4, (a40, a41, a42)),
                       (ix5, (a50, a51))):
        for r, col in enumerate(cols_):
            pltpu.sync_copy(col.at[pl.ds(b0, per_w)], ref.at[r])
    for r, col in enumerate((a60, a61, a62)):
        pltpu.sync_copy(col.at[pl.ds(bn, per_w_neg)], ix6.at[r])

    def make_term(ixref, crows, rel_row, nchunks, compute):
        def fire(c, slot):
            for k, rrow in enumerate(crows):
                pltpu.async_copy(
                    cls_hbm.at[ixref.at[rrow, pl.ds(c * CHUNK, CHUNK)]],
                    (ca, cb, ce)[k].at[slot], sems[slot])
            if rel_row is not None:
                pltpu.async_copy(
                    rel_hbm.at[ixref.at[rel_row, pl.ds(c * CHUNK, CHUNK)]],
                    rb.at[slot], sems[slot])

        def drain(c, slot):
            for k, rrow in enumerate(crows):
                pltpu.make_async_copy(
                    cls_hbm.at[ixref.at[rrow, pl.ds(c * CHUNK, CHUNK)]],
                    (ca, cb, ce)[k].at[slot], sems[slot]).wait()
            if rel_row is not None:
                pltpu.make_async_copy(
                    rel_hbm.at[ixref.at[rel_row, pl.ds(c * CHUNK, CHUNK)]],
                    rb.at[slot], sems[slot]).wait()

        def run(acc):
            fire(0, 0)

            def pair(p, acc):
                c0 = 2 * p
                drain(c0, 0)
                fire(c0 + 1, 1)
                acc = compute(0, acc)
                drain(c0 + 1, 1)

                @pl.when(c0 + 2 < nchunks)
                def _():
                    fire(c0 + 2, 0)
                return compute(1, acc)
            return lax.fori_loop(0, nchunks // 2, pair, acc)
        return run

    def items_body(slot, acc4, fn):
        def item(i2, acc4):
            a = list(acc4)
            for u in range(2):
                i = i2 * 2 + u
                for j in range(8):
                    t = fn(slot, i, j)
                    k = (u * 8 + j) % 4
                    a[k] = a[k] + t * t
            return tuple(a)
        return lax.fori_loop(0, CHUNK // 2, item, acc4)

    def f_t1(slot, i, j):
        c1 = ca[slot, i, pl.ds(j * 16, 16)]
        c2 = cb[slot, i, pl.ds(j * 16, 16)]
        o1 = jnp.abs(ca[slot, i, pl.ds(DIM + j * 16, 16)])
        o2 = jnp.abs(cb[slot, i, pl.ds(DIM + j * 16, 16)])
        return _relu(jnp.abs(c1 - c2) + o1 - o2)

    def f_t3(slot, i, j):
        c1 = ca[slot, i, pl.ds(j * 16, 16)]
        c2 = cb[slot, i, pl.ds(j * 16, 16)]
        o1 = jnp.abs(ca[slot, i, pl.ds(DIM + j * 16, 16)])
        o2 = jnp.abs(cb[slot, i, pl.ds(DIM + j * 16, 16)])
        r = rb[slot, i, pl.ds(j * 16, 16)]
        return _relu(jnp.abs(c1 + r - c2) + o1 - o2)

    def f_t4(slot, i, j):
        c1 = ca[slot, i, pl.ds(j * 16, 16)]
        c2 = cb[slot, i, pl.ds(j * 16, 16)]
        o1 = jnp.abs(ca[slot, i, pl.ds(DIM + j * 16, 16)])
        o2 = jnp.abs(cb[slot, i, pl.ds(DIM + j * 16, 16)])
        r = rb[slot, i, pl.ds(j * 16, 16)]
        return _relu(jnp.abs(c1 - r - c2) + o1 - o2)

    def f_t5(slot, i, j):
        c1 = ca[slot, i, pl.ds(j * 16, 16)]
        c2 = cb[slot, i, pl.ds(j * 16, 16)]
        o1 = jnp.abs(ca[slot, i, pl.ds(DIM + j * 16, 16)])
        o2 = jnp.abs(cb[slot, i, pl.ds(DIM + j * 16, 16)])
        return _relu(o1 + o2 - jnp.abs(c1 - c2))

    def t2_compute(slot, acc):
        def group(g, acc):
            rows = g * 16 + lane

            def dims(d0, carry):
                sa0, sa1, sb0, sb1 = carry
                for dd in range(4):
                    d = d0 * 4 + dd
                    cd = jnp.full((16,), d, jnp.int32)
                    od = jnp.full((16,), DIM + d, jnp.int32)
                    c1 = plsc.load_gather(ca.at[slot], [rows, cd])
                    o1 = jnp.abs(plsc.load_gather(ca.at[slot], [rows, od]))
                    c2 = plsc.load_gather(cb.at[slot], [rows, cd])
                    o2 = jnp.abs(plsc.load_gather(cb.at[slot], [rows, od]))
                    ec = plsc.load_gather(ce.at[slot], [rows, cd])
                    eo = jnp.abs(plsc.load_gather(ce.at[slot], [rows, od]))
                    lower = jnp.maximum(c1 - o1, c2 - o2)
                    upper = jnp.minimum(c1 + o1, c2 + o2)
                    icn = (lower + upper) * 0.5
                    io = jnp.abs(upper - lower) * 0.5
                    t = _relu(jnp.abs(icn - ec) + io - eo)
                    u = _relu(lower - upper)
                    if dd % 2 == 0:
                        sa0 = sa0 + t * t
                        sb0 = sb0 + u * u
                    else:
                        sa1 = sa1 + t * t
                        sb1 = sb1 + u * u
                return (sa0, sa1, sb0, sb1)

            sa0, sa1, sb0, sb1 = lax.fori_loop(0, DIM // 4, dims,
                                               (f0, f0, f0, f0))
            sa = sa0 + sa1
            sb = sb0 + sb1
            return acc + sa + sb + 2.0 * _vsqrt(sa * sb)
        return lax.fori_loop(0, CHUNK // 16, group, acc)

    def t6_compute(slot, accs):
        def group(g, accs):
            a6a, a6b = accs
            rows = g * 16 + lane

            def dims(d0, carry):
                s0, s1 = carry
                for dd in range(4):
                    d = d0 * 4 + dd
                    cd = jnp.full((16,), d, jnp.int32)
                    od = jnp.full((16,), DIM + d, jnp.int32)
                    c1 = plsc.load_gather(ca.at[slot], [rows, cd])
                    o1 = jnp.abs(plsc.load_gather(ca.at[slot], [rows, od]))
                    c2 = plsc.load_gather(cb.at[slot], [rows, cd])
                    o2 = jnp.abs(plsc.load_gather(cb.at[slot], [rows, od]))
                    r = plsc.load_gather(rb.at[slot], [rows, cd])
                    t = _relu(jnp.abs(c1 + r - c2) - o1 - o2)
                    if dd % 2 == 0:
                        s0 = s0 + t * t
                    else:
                        s1 = s1 + t * t
                return (s0, s1)

            s0, s1 = lax.fori_loop(0, DIM // 4, dims, (f0, f0))
            sv = s0 + s1
            return (a6a + _vsqrt(sv), a6b + sv)
        return lax.fori_loop(0, CHUNK // 16, group, accs)

    z4 = (f0, f0, f0, f0)
    t1 = make_term(ix1, (0, 1), None, nch,
                   lambda s, a: items_body(s, a, f_t1))(z4)
    t2 = make_term(ix2, (0, 1, 2), None, nch, t2_compute)(f0)
    t3 = make_term(ix3, (0, 2), 1, nch,
                   lambda s, a: items_body(s, a, f_t3))(z4)
    t4 = make_term(ix4, (1, 2), 0, nch,
                   lambda s, a: items_body(s, a, f_t4))(z4)
    t5 = make_term(ix5, (0, 1), None, nch,
                   lambda s, a: items_body(s, a, f_t5))(z4)
    t6a, t6b = make_term(ix6, (0, 2), 1, nch_neg, t6_compute)((f0, f0))

    def red4(a):
        return (a[0] + a[1]) + (a[2] + a[3])

    for k, val in enumerate((red4(t1), t2, red4(t3), red4(t4), red4(t5),
                             t6a, t6b, f0)):
        outv[k, :] = val
    pltpu.sync_copy(outv, out_hbm.at[wid])


def _sc_partials(cls_e, rel_e, cols):
    b = cols[0].shape[0]
    per_w = b // NW
    f = functools.partial(
        pl.kernel,
        mesh=plsc.VectorSubcoreMesh(core_axis_name="c", subcore_axis_name="s"),
        compiler_params=pltpu.CompilerParams(
            needs_layout_passes=False, use_tc_tiling_on_sc=False),
        out_type=jax.ShapeDtypeStruct((NW, 8, 16), jnp.float32),
        scratch_types=[
            pltpu.VMEM((2, per_w), jnp.int32),
            pltpu.VMEM((3, per_w), jnp.int32),
            pltpu.VMEM((3, per_w), jnp.int32),
            pltpu.VMEM((3, per_w), jnp.int32),
            pltpu.VMEM((2, per_w), jnp.int32),
            pltpu.VMEM((3, 2 * per_w), jnp.int32),
            pltpu.VMEM((2, CHUNK, NCOLS), jnp.float32),
            pltpu.VMEM((2, CHUNK, NCOLS), jnp.float32),
            pltpu.VMEM((2, CHUNK, NCOLS), jnp.float32),
            pltpu.VMEM((2, CHUNK, DIM), jnp.float32),
            pltpu.VMEM((8, 16), jnp.float32),
            pltpu.SemaphoreType.DMA,
            pltpu.SemaphoreType.DMA,
        ],
    )(_sc_body)
    return f(cls_e, rel_e, *cols)


def kernel(nf1, nf2, nf3, nf4, disjoint, nf3_neg0, nf3_neg1,
           class_embeds, relation_embeds):
    i32 = jnp.int32
    neg = jnp.concatenate([nf3_neg0, nf3_neg1], axis=0)
    cols = [
        nf1[:, 0], nf1[:, 1],
        nf2[:, 0], nf2[:, 1], nf2[:, 2],
        nf3[:, 0], nf3[:, 1], nf3[:, 2],
        nf4[:, 0], nf4[:, 1], nf4[:, 2],
        disjoint[:, 0], disjoint[:, 1],
        neg[:, 0], neg[:, 1], neg[:, 2],
    ]
    cols = [jnp.asarray(c, dtype=i32) for c in cols]
    part = _sc_partials(class_embeds, relation_embeds, cols)
    s = jnp.sum(part, axis=(0, 2))
    b = nf1.shape[0]
    loss = (s[0] + s[1] + s[2] + s[3] + s[4]) / b \
        + 4.0 + (s[6] - 4.0 * s[5]) / (2 * b)
    return loss.astype(jnp.float32)


# speedup vs baseline: 2.6385x; 2.6385x over previous
import functools

import jax
import jax.numpy as jnp
from jax import lax
from jax.experimental import pallas as pl
from jax.experimental.pallas import tpu as pltpu
from jax.experimental.pallas import tpu_sc as plsc

DIM = 128
NCOLS = 2 * DIM
NC = 2
NS = 16
NW = NC * NS
CHUNK = 64


def _vsqrt(x):
    i = lax.bitcast_convert_type(x, jnp.int32)
    y = lax.bitcast_convert_type(0x5F3759DF - (i >> 1), jnp.float32)
    for _ in range(3):
        y = y * (1.5 - 0.5 * x * y * y)
    return x * y


def _relu(v):
    return jnp.maximum(v, 0.0)


def _sc_body(cls_hbm, rel_hbm,
             a10, a11,
             a20, a21, a22,
             a30, a31, a32,
             a40, a41, a42,
             a50, a51,
             a60, a61, a62,
             out_hbm,
             ix1, ix2, ix3, ix4, ix5, ix6,
             ca, cb, ce, rb, outv, sem0, sem1):
    wid = lax.axis_index("s") * NC + lax.axis_index("c")
    n_items = a10.shape[0]
    per_w = n_items // NW
    per_w_neg = a60.shape[0] // NW
    nch = per_w // CHUNK
    nch_neg = per_w_neg // CHUNK
    f0 = jnp.zeros((16,), jnp.float32)
    lane = lax.iota(jnp.int32, 16)
    sems = (sem0, sem1)

    b0 = wid * per_w
    bn = wid * per_w_neg
    for ref, cols_ in ((ix1, (a10, a11)), (ix2, (a20, a21, a22)),
                       (ix3, (a30, a31, a32)), (ix4, (a40, a41, a42)),
                       (ix5, (a50, a51))):
        for r, col in enumerate(cols_):
            pltpu.sync_copy(col.at[pl.ds(b0, per_w)], ref.at[r])
    for r, col in enumerate((a60, a61, a62)):
        pltpu.sync_copy(col.at[pl.ds(bn, per_w_neg)], ix6.at[r])

    def make_term(ixref, crows, rel_row, nchunks, compute):
        def fire(c, slot):
            for k, rrow in enumerate(crows):
                pltpu.async_copy(
                    cls_hbm.at[ixref.at[rrow, pl.ds(c * CHUNK, CHUNK)]],
                    (ca, cb, ce)[k].at[slot], sems[slot])
            if rel_row is not None:
                pltpu.async_copy(
                    rel_hbm.at[ixref.at[rel_row, pl.ds(c * CHUNK, CHUNK)]],
                    rb.at[slot], sems[slot])

        def drain(c, slot):
            for k, rrow in enumerate(crows):
                pltpu.make_async_copy(
                    cls_hbm.at[ixref.at[rrow, pl.ds(c * CHUNK, CHUNK)]],
                    (ca, cb, ce)[k].at[slot], sems[slot]).wait()
            if rel_row is not None:
                pltpu.make_async_copy(
                    rel_hbm.at[ixref.at[rel_row, pl.ds(c * CHUNK, CHUNK)]],
                    rb.at[slot], sems[slot]).wait()

        def run(acc):
            fire(0, 0)

            def pair(p, acc):
                c0 = 2 * p
                drain(c0, 0)
                fire(c0 + 1, 1)
                acc = compute(0, acc)
                drain(c0 + 1, 1)

                @pl.when(c0 + 2 < nchunks)
                def _():
                    fire(c0 + 2, 0)
                return compute(1, acc)
            return lax.fori_loop(0, nchunks // 2, pair, acc)
        return run

    def items_body(slot, acc4, fn):
        def item(i2, acc4):
            a = list(acc4)
            for u in range(2):
                i = i2 * 2 + u
                for j in range(8):
                    t = fn(slot, i, j)
                    k = (u * 8 + j) % 4
                    a[k] = a[k] + t * t
            return tuple(a)
        return lax.fori_loop(0, CHUNK // 2, item, acc4)

    def f_t1(slot, i, j):
        c1 = ca[slot, i, pl.ds(j * 16, 16)]
        c2 = cb[slot, i, pl.ds(j * 16, 16)]
        o1 = jnp.abs(ca[slot, i, pl.ds(DIM + j * 16, 16)])
        o2 = jnp.abs(cb[slot, i, pl.ds(DIM + j * 16, 16)])
        return _relu(jnp.abs(c1 - c2) + o1 - o2)

    def f_t3(slot, i, j):
        c1 = ca[slot, i, pl.ds(j * 16, 16)]
        c2 = cb[slot, i, pl.ds(j * 16, 16)]
        o1 = jnp.abs(ca[slot, i, pl.ds(DIM + j * 16, 16)])
        o2 = jnp.abs(cb[slot, i, pl.ds(DIM + j * 16, 16)])
        r = rb[slot, i, pl.ds(j * 16, 16)]
        return _relu(jnp.abs(c1 + r - c2) + o1 - o2)

    def f_t4(slot, i, j):
        c1 = ca[slot, i, pl.ds(j * 16, 16)]
        c2 = cb[slot, i, pl.ds(j * 16, 16)]
        o1 = jnp.abs(ca[slot, i, pl.ds(DIM + j * 16, 16)])
        o2 = jnp.abs(cb[slot, i, pl.ds(DIM + j * 16, 16)])
        r = rb[slot, i, pl.ds(j * 16, 16)]
        return _relu(jnp.abs(c1 - r - c2) + o1 - o2)

    def f_t5(slot, i, j):
        c1 = ca[slot, i, pl.ds(j * 16, 16)]
        c2 = cb[slot, i, pl.ds(j * 16, 16)]
        o1 = jnp.abs(ca[slot, i, pl.ds(DIM + j * 16, 16)])
        o2 = jnp.abs(cb[slot, i, pl.ds(DIM + j * 16, 16)])
        return _relu(o1 + o2 - jnp.abs(c1 - c2))

    def _lanesum(v):
        for sh in (8, 4, 2, 1):
            v = v + v.at[lane ^ sh].get(mode="promise_in_bounds")
        return v

    def t2_compute(slot, acc):
        def item(i, acc):
            sa0 = sa1 = sb0 = sb1 = f0
            for j in range(8):
                c1 = ca[slot, i, pl.ds(j * 16, 16)]
                o1 = jnp.abs(ca[slot, i, pl.ds(DIM + j * 16, 16)])
                c2 = cb[slot, i, pl.ds(j * 16, 16)]
                o2 = jnp.abs(cb[slot, i, pl.ds(DIM + j * 16, 16)])
                ec = ce[slot, i, pl.ds(j * 16, 16)]
                eo = jnp.abs(ce[slot, i, pl.ds(DIM + j * 16, 16)])
                lower = jnp.maximum(c1 - o1, c2 - o2)
                upper = jnp.minimum(c1 + o1, c2 + o2)
                t = _relu(jnp.abs(lower + upper - (ec + ec))
                          + jnp.abs(upper - lower) - (eo + eo))
                u = _relu(lower - upper)
                if j % 2 == 0:
                    sa0 = sa0 + t * t
                    sb0 = sb0 + u * u
                else:
                    sa1 = sa1 + t * t
                    sb1 = sb1 + u * u
            sa = _lanesum(sa0 + sa1)
            sb = _lanesum(sb0 + sb1)
            return acc + 0.25 * sa + sb + _vsqrt(sa * sb)
        return lax.fori_loop(0, CHUNK, item, acc)

    def t6_compute(slot, accs):
        def item(i, accs):
            a6a, a6b = accs
            s0 = s1 = f0
            for j in range(8):
                c1 = ca[slot, i, pl.ds(j * 16, 16)]
                o1 = jnp.abs(ca[slot, i, pl.ds(DIM + j * 16, 16)])
                c2 = cb[slot, i, pl.ds(j * 16, 16)]
                o2 = jnp.abs(cb[slot, i, pl.ds(DIM + j * 16, 16)])
                r = rb[slot, i, pl.ds(j * 16, 16)]
                t = _relu(jnp.abs(c1 + r - c2) - o1 - o2)
                if j % 2 == 0:
                    s0 = s0 + t * t
                else:
                    s1 = s1 + t * t
            sv = _lanesum(s0 + s1)
            return (a6a + _vsqrt(sv), a6b + sv)
        return lax.fori_loop(0, CHUNK, item, accs)

    z4 = (f0, f0, f0, f0)
    t1 = make_term(ix1, (0, 1), None, nch,
                   lambda s, a: items_body(s, a, f_t1))(z4)
    t2 = make_term(ix2, (0, 1, 2), None, nch, t2_compute)(f0)
    t3 = make_term(ix3, (0, 2), 1, nch,
                   lambda s, a: items_body(s, a, f_t3))(z4)
    t4 = make_term(ix4, (1, 2), 0, nch,
                   lambda s, a: items_body(s, a, f_t4))(z4)
    t5 = make_term(ix5, (0, 1), None, nch,
                   lambda s, a: items_body(s, a, f_t5))(z4)
    t6a, t6b = make_term(ix6, (0, 2), 1, nch_neg, t6_compute)((f0, f0))

    def red4(a):
        return (a[0] + a[1]) + (a[2] + a[3])

    for k, val in enumerate((red4(t1), t2, red4(t3), red4(t4), red4(t5),
                             t6a, t6b, f0)):
        outv[k, :] = val
    pltpu.sync_copy(outv, out_hbm.at[wid])


def _sc_partials(cls_e, rel_e, cols):
    b = cols[0].shape[0]
    per_w = b // NW
    f = functools.partial(
        pl.kernel,
        mesh=plsc.VectorSubcoreMesh(core_axis_name="c", subcore_axis_name="s"),
        compiler_params=pltpu.CompilerParams(
            needs_layout_passes=False, use_tc_tiling_on_sc=False),
        out_type=jax.ShapeDtypeStruct((NW, 8, 16), jnp.float32),
        scratch_types=[
            pltpu.VMEM((2, per_w), jnp.int32),
            pltpu.VMEM((3, per_w), jnp.int32),
            pltpu.VMEM((3, per_w), jnp.int32),
            pltpu.VMEM((3, per_w), jnp.int32),
            pltpu.VMEM((2, per_w), jnp.int32),
            pltpu.VMEM((3, 2 * per_w), jnp.int32),
            pltpu.VMEM((2, CHUNK, NCOLS), jnp.float32),
            pltpu.VMEM((2, CHUNK, NCOLS), jnp.float32),
            pltpu.VMEM((2, CHUNK, NCOLS), jnp.float32),
            pltpu.VMEM((2, CHUNK, DIM), jnp.float32),
            pltpu.VMEM((8, 16), jnp.float32),
            pltpu.SemaphoreType.DMA,
            pltpu.SemaphoreType.DMA,
        ],
    )(_sc_body)
    return f(cls_e, rel_e, *cols)


def kernel(nf1, nf2, nf3, nf4, disjoint, nf3_neg0, nf3_neg1,
           class_embeds, relation_embeds):
    i32 = jnp.int32
    neg = jnp.concatenate([nf3_neg0, nf3_neg1], axis=0)
    cols = [
        nf1[:, 0], nf1[:, 1],
        nf2[:, 0], nf2[:, 1], nf2[:, 2],
        nf3[:, 0], nf3[:, 1], nf3[:, 2],
        nf4[:, 0], nf4[:, 1], nf4[:, 2],
        disjoint[:, 0], disjoint[:, 1],
        neg[:, 0], neg[:, 1], neg[:, 2],
    ]
    cols = [jnp.asarray(c, dtype=i32) for c in cols]
    part = _sc_partials(class_embeds, relation_embeds, cols)
    s = jnp.sum(part, axis=(0, 2))
    b = nf1.shape[0]
    loss = (s[0] + s[1] / 16.0 + s[2] + s[3] + s[4]) / b \
        + 4.0 + (s[6] / 16.0 - 4.0 * s[5] / 16.0) / (2 * b)
    return loss.astype(jnp.float32)


# speedup vs baseline: 3.5271x; 1.3368x over previous
import functools

import jax
import jax.numpy as jnp
from jax import lax
from jax.experimental import pallas as pl
from jax.experimental.pallas import tpu as pltpu
from jax.experimental.pallas import tpu_sc as plsc

DIM = 128
NCOLS = 2 * DIM
NC = 2
NS = 16
NW = NC * NS
CHUNK = 64


def _vsqrt(x):
    i = lax.bitcast_convert_type(x, jnp.int32)
    y = lax.bitcast_convert_type(0x5F3759DF - (i >> 1), jnp.float32)
    for _ in range(3):
        y = y * (1.5 - 0.5 * x * y * y)
    return x * y


def _relu(v):
    return jnp.maximum(v, 0.0)


def _sc_body(cls_hbm, rel_hbm,
             a10, a11,
             a20, a21, a22,
             a30, a31, a32,
             a40, a41, a42,
             a50, a51,
             a60, a61, a62,
             out_hbm,
             x10, x11, x20, x21, x22, x30, x31, x32,
             x40, x41, x42, x50, x51, x60, x61, x62,
             ca0, ca1, cb0, cb1, ce0, ce1, rb0, rb1,
             outv, sem0, sem1):
    wid = lax.axis_index("s") * NC + lax.axis_index("c")
    n_items = a10.shape[0]
    per_w = n_items // NW
    per_w_neg = a60.shape[0] // NW
    nch = per_w // CHUNK
    nch_neg = per_w_neg // CHUNK
    f0 = jnp.zeros((16,), jnp.float32)
    lane = lax.iota(jnp.int32, 16)
    sems = (sem0, sem1)
    cas = (ca0, ca1)
    cbs = (cb0, cb1)
    ces = (ce0, ce1)
    rbs = (rb0, rb1)

    b0 = wid * per_w
    bn = wid * per_w_neg
    for xref, col in ((x10, a10), (x11, a11), (x20, a20), (x21, a21),
                      (x22, a22), (x30, a30), (x31, a31), (x32, a32),
                      (x40, a40), (x41, a41), (x42, a42), (x50, a50),
                      (x51, a51)):
        pltpu.sync_copy(col.at[pl.ds(b0, per_w)], xref)
    for xref, col in ((x60, a60), (x61, a61), (x62, a62)):
        pltpu.sync_copy(col.at[pl.ds(bn, per_w_neg)], xref)

    def make_term(xcols, rel_x, nchunks, compute):
        H = CHUNK // 2

        def fire(c, slot):
            for k, x in enumerate(xcols):
                dst = (cas, cbs, ces)[k][slot]
                for h in range(2):
                    pltpu.async_copy(
                        cls_hbm.at[x.at[pl.ds(c * CHUNK + h * H, H)]],
                        dst.at[pl.ds(h * H, H)], sems[slot])
            if rel_x is not None:
                for h in range(2):
                    pltpu.async_copy(
                        rel_hbm.at[rel_x.at[pl.ds(c * CHUNK + h * H, H)]],
                        rbs[slot].at[pl.ds(h * H, H)], sems[slot])

        def drain(c, slot):
            for k, x in enumerate(xcols):
                dst = (cas, cbs, ces)[k][slot]
                for h in range(2):
                    pltpu.make_async_copy(
                        cls_hbm.at[x.at[pl.ds(c * CHUNK + h * H, H)]],
                        dst.at[pl.ds(h * H, H)], sems[slot]).wait()
            if rel_x is not None:
                for h in range(2):
                    pltpu.make_async_copy(
                        rel_hbm.at[rel_x.at[pl.ds(c * CHUNK + h * H, H)]],
                        rbs[slot].at[pl.ds(h * H, H)], sems[slot]).wait()

        def run(acc):
            fire(0, 0)

            def pair(p, acc):
                c0 = 2 * p
                drain(c0, 0)
                fire(c0 + 1, 1)
                acc = compute(0, acc)
                drain(c0 + 1, 1)

                @pl.when(c0 + 2 < nchunks)
                def _():
                    fire(c0 + 2, 0)
                return compute(1, acc)
            return lax.fori_loop(0, nchunks // 2, pair, acc)
        return run

    def items_body(slot, acc4, fn):
        def item(i2, acc4):
            a = list(acc4)
            for u in range(2):
                i = i2 * 2 + u
                for j in range(8):
                    t = fn(slot, i, j)
                    k = (u * 8 + j) % 4
                    a[k] = a[k] + t * t
            return tuple(a)
        return lax.fori_loop(0, CHUNK // 2, item, acc4)

    def f_t1(slot, i, j):
        c1 = cas[slot][i, pl.ds(j * 16, 16)]
        c2 = cbs[slot][i, pl.ds(j * 16, 16)]
        o1 = jnp.abs(cas[slot][i, pl.ds(DIM + j * 16, 16)])
        o2 = jnp.abs(cbs[slot][i, pl.ds(DIM + j * 16, 16)])
        return _relu(jnp.abs(c1 - c2) + o1 - o2)

    def f_t3(slot, i, j):
        c1 = cas[slot][i, pl.ds(j * 16, 16)]
        c2 = cbs[slot][i, pl.ds(j * 16, 16)]
        o1 = jnp.abs(cas[slot][i, pl.ds(DIM + j * 16, 16)])
        o2 = jnp.abs(cbs[slot][i, pl.ds(DIM + j * 16, 16)])
        r = rbs[slot][i, pl.ds(j * 16, 16)]
        return _relu(jnp.abs(c1 + r - c2) + o1 - o2)

    def f_t4(slot, i, j):
        c1 = cas[slot][i, pl.ds(j * 16, 16)]
        c2 = cbs[slot][i, pl.ds(j * 16, 16)]
        o1 = jnp.abs(cas[slot][i, pl.ds(DIM + j * 16, 16)])
        o2 = jnp.abs(cbs[slot][i, pl.ds(DIM + j * 16, 16)])
        r = rbs[slot][i, pl.ds(j * 16, 16)]
        return _relu(jnp.abs(c1 - r - c2) + o1 - o2)

    def f_t5(slot, i, j):
        c1 = cas[slot][i, pl.ds(j * 16, 16)]
        c2 = cbs[slot][i, pl.ds(j * 16, 16)]
        o1 = jnp.abs(cas[slot][i, pl.ds(DIM + j * 16, 16)])
        o2 = jnp.abs(cbs[slot][i, pl.ds(DIM + j * 16, 16)])
        return _relu(o1 + o2 - jnp.abs(c1 - c2))

    def _lanesum(v):
        for sh in (8, 4, 2, 1):
            v = v + v.at[lane ^ sh].get(mode="promise_in_bounds")
        return v

    def t2_compute(slot, acc):
        def item(i, acc):
            sa0 = sa1 = sb0 = sb1 = f0
            for j in range(8):
                c1 = cas[slot][i, pl.ds(j * 16, 16)]
                o1 = jnp.abs(cas[slot][i, pl.ds(DIM + j * 16, 16)])
                c2 = cbs[slot][i, pl.ds(j * 16, 16)]
                o2 = jnp.abs(cbs[slot][i, pl.ds(DIM + j * 16, 16)])
                ec = ces[slot][i, pl.ds(j * 16, 16)]
                eo = jnp.abs(ces[slot][i, pl.ds(DIM + j * 16, 16)])
                lower = jnp.maximum(c1 - o1, c2 - o2)
                upper = jnp.minimum(c1 + o1, c2 + o2)
                t = _relu(jnp.abs(lower + upper - (ec + ec))
                          + jnp.abs(upper - lower) - (eo + eo))
                u = _relu(lower - upper)
                if j % 2 == 0:
                    sa0 = sa0 + t * t
                    sb0 = sb0 + u * u
                else:
                    sa1 = sa1 + t * t
                    sb1 = sb1 + u * u
            sa = _lanesum(sa0 + sa1)
            sb = _lanesum(sb0 + sb1)
            return acc + 0.25 * sa + sb + _vsqrt(sa * sb)
        return lax.fori_loop(0, CHUNK, item, acc)

    def t6_compute(slot, accs):
        def item(i, accs):
            a6a, a6b = accs
            s0 = s1 = f0
            for j in range(8):
                c1 = cas[slot][i, pl.ds(j * 16, 16)]
                o1 = jnp.abs(cas[slot][i, pl.ds(DIM + j * 16, 16)])
                c2 = cbs[slot][i, pl.ds(j * 16, 16)]
                o2 = jnp.abs(cbs[slot][i, pl.ds(DIM + j * 16, 16)])
                r = rbs[slot][i, pl.ds(j * 16, 16)]
                t = _relu(jnp.abs(c1 + r - c2) - o1 - o2)
                if j % 2 == 0:
                    s0 = s0 + t * t
                else:
                    s1 = s1 + t * t
            sv = _lanesum(s0 + s1)
            return (a6a + _vsqrt(sv), a6b + sv)
        return lax.fori_loop(0, CHUNK, item, accs)

    z4 = (f0, f0, f0, f0)
    t1 = make_term((x10, x11), None, nch,
                   lambda s, a: items_body(s, a, f_t1))(z4)
    t2 = make_term((x20, x21, x22), None, nch, t2_compute)(f0)
    t3 = make_term((x30, x32), x31, nch,
                   lambda s, a: items_body(s, a, f_t3))(z4)
    t4 = make_term((x41, x42), x40, nch,
                   lambda s, a: items_body(s, a, f_t4))(z4)
    t5 = make_term((x50, x51), None, nch,
                   lambda s, a: items_body(s, a, f_t5))(z4)
    t6a, t6b = make_term((x60, x62), x61, nch_neg, t6_compute)((f0, f0))

    def red4(a):
        return (a[0] + a[1]) + (a[2] + a[3])

    for k, val in enumerate((red4(t1), t2, red4(t3), red4(t4), red4(t5),
                             t6a, t6b, f0)):
        outv[pl.ds(k * 16, 16)] = val
    pltpu.sync_copy(outv, out_hbm.at[wid])


def _sc_partials(cls_e, rel_e, cols):
    b = cols[0].shape[0]
    per_w = b // NW
    f = functools.partial(
        pl.kernel,
        mesh=plsc.VectorSubcoreMesh(core_axis_name="c", subcore_axis_name="s"),
        compiler_params=pltpu.CompilerParams(needs_layout_passes=False),
        out_type=jax.ShapeDtypeStruct((NW, 128), jnp.float32),
        scratch_types=(
            [pltpu.VMEM((per_w,), jnp.int32) for _ in range(13)]
            + [pltpu.VMEM((2 * per_w,), jnp.int32) for _ in range(3)]
            + [pltpu.VMEM((CHUNK, NCOLS), jnp.float32) for _ in range(6)]
            + [pltpu.VMEM((CHUNK, DIM), jnp.float32) for _ in range(2)]
            + [pltpu.VMEM((128,), jnp.float32),
               pltpu.SemaphoreType.DMA,
               pltpu.SemaphoreType.DMA]
        ),
    )(_sc_body)
    return f(cls_e, rel_e, *cols)


def kernel(nf1, nf2, nf3, nf4, disjoint, nf3_neg0, nf3_neg1,
           class_embeds, relation_embeds):
    i32 = jnp.int32
    neg = jnp.concatenate([nf3_neg0, nf3_neg1], axis=0)
    cols = [
        nf1[:, 0], nf1[:, 1],
        nf2[:, 0], nf2[:, 1], nf2[:, 2],
        nf3[:, 0], nf3[:, 1], nf3[:, 2],
        nf4[:, 0], nf4[:, 1], nf4[:, 2],
        disjoint[:, 0], disjoint[:, 1],
        neg[:, 0], neg[:, 1], neg[:, 2],
    ]
    cols = [jnp.asarray(c, dtype=i32) for c in cols]
    part = _sc_partials(class_embeds, relation_embeds, cols)
    s = jnp.sum(part.reshape(NW, 8, 16), axis=(0, 2))
    b = nf1.shape[0]
    loss = (s[0] + s[1] / 16.0 + s[2] + s[3] + s[4]) / b \
        + 4.0 + (s[6] / 16.0 - 4.0 * s[5] / 16.0) / (2 * b)
    return loss.astype(jnp.float32)
